# baseline (speedup 1.0000x reference)
import functools

import jax
import jax.numpy as jnp
from jax import lax
from jax.experimental import pallas as pl
from jax.experimental.pallas import tpu as pltpu
from jax.experimental.pallas import tpu_sc as plsc

VOCAB = 1000000
DIM = 32
B = 16384
ALPHA = 0.75
X_MAX = 100.0

_NC = 2
_NS = 16
_NW = _NC * _NS
_BPW = B // _NW


def _sc_gather_body(i_hbm, j_hbm, w_hbm, c_hbm, wb_hbm, cb_hbm,
                    w_out, c_out, wb_out, cb_out,
                    idx_i, idx_j, w_v, c_v, wb_v, cb_v, sem):
    wid = lax.axis_index("s") * _NC + lax.axis_index("c")
    base = wid * _BPW
    pltpu.sync_copy(i_hbm.at[pl.ds(base, _BPW)], idx_i)
    pltpu.sync_copy(j_hbm.at[pl.ds(base, _BPW)], idx_j)
    cp1 = pltpu.async_copy(w_hbm.at[idx_i], w_v, sem)
    cp2 = pltpu.async_copy(c_hbm.at[idx_j], c_v, sem)
    cp3 = pltpu.async_copy(wb_hbm.at[idx_i], wb_v, sem)
    cp4 = pltpu.async_copy(cb_hbm.at[idx_j], cb_v, sem)
    cp1.wait()
    cp2.wait()
    cp3.wait()
    cp4.wait()
    pltpu.sync_copy(w_v, w_out.at[pl.ds(base, _BPW)])
    pltpu.sync_copy(c_v, c_out.at[pl.ds(base, _BPW)])
    pltpu.sync_copy(wb_v, wb_out.at[pl.ds(base, _BPW)])
    pltpu.sync_copy(cb_v, cb_out.at[pl.ds(base, _BPW)])


_sc_gather = functools.partial(
    pl.kernel,
    mesh=plsc.VectorSubcoreMesh(core_axis_name="c", subcore_axis_name="s"),
    out_type=[
        jax.ShapeDtypeStruct((B, DIM), jnp.float32),
        jax.ShapeDtypeStruct((B, DIM), jnp.float32),
        jax.ShapeDtypeStruct((B,), jnp.float32),
        jax.ShapeDtypeStruct((B,), jnp.float32),
    ],
    scratch_types=[
        pltpu.VMEM((_BPW,), jnp.int32),
        pltpu.VMEM((_BPW,), jnp.int32),
        pltpu.VMEM((_BPW, DIM), jnp.float32),
        pltpu.VMEM((_BPW, DIM), jnp.float32),
        pltpu.VMEM((_BPW,), jnp.float32),
        pltpu.VMEM((_BPW,), jnp.float32),
        pltpu.SemaphoreType.DMA,
    ],
    compiler_params=pltpu.CompilerParams(use_tc_tiling_on_sc=False),
)(_sc_gather_body)


_TC_BLK = 2048


def _tc_loss_body(x_ref, w_ref, c_ref, wb_ref, cb_ref, o_ref):
    w = w_ref[...]
    c = c_ref[...]
    s = jnp.sum(w * c, axis=1)
    x = x_ref[...]
    f = jnp.where(x < X_MAX, (x * (1.0 / X_MAX)) ** ALPHA, jnp.float32(1.0))
    o_ref[...] = f * (s + wb_ref[...] + cb_ref[...] - jnp.log(x))


def _tc_loss(x, w_rows, c_rows, wb, cb):
    grid = (B // _TC_BLK,)
    vec_spec = pl.BlockSpec((_TC_BLK,), lambda k: (k,))
    row_spec = pl.BlockSpec((_TC_BLK, DIM), lambda k: (k, 0))
    return pl.pallas_call(
        _tc_loss_body,
        grid=grid,
        in_specs=[vec_spec, row_spec, row_spec, vec_spec, vec_spec],
        out_specs=vec_spec,
        out_shape=jax.ShapeDtypeStruct((B,), jnp.float32),
    )(x, w_rows, c_rows, wb, cb)


def kernel(x, i, j, w_table, c_table, w_bias, c_bias):
    i32 = i.astype(jnp.int32)
    j32 = j.astype(jnp.int32)
    w_rows, c_rows, wb, cb = _sc_gather(i32, j32, w_table, c_table,
                                        w_bias, c_bias)
    return _tc_loss(x, w_rows, c_rows, wb, cb)
